# baseline (speedup 1.0000x reference)
import jax
import jax.numpy as jnp
import numpy as np
from jax import lax
from jax.experimental import pallas as pl
from jax.experimental.pallas import tpu as pltpu
from jax.experimental.pallas import tpu_sc as plsc

F32 = jnp.float32
BF16 = jnp.bfloat16

N = 10000
E = 320000
D = 128
NHEADS = 4
HDIM = 32
NGRAPH = 64

NCORES = 2
NSUB = 16
NW = NCORES * NSUB
SLICES = (192000, 128000)
CB = 80
NPAD = 10240
RPT = NPAD // NSUB

_SIGMA = np.zeros(D, np.int32)
for _m in range(D // 32):
    for _i in range(16):
        for _b in range(2):
            _SIGMA[32 * _m + 2 * _i + _b] = 32 * _m + 16 * _b + _i


def _pre_body(x_ref, waT_ref, wbT_ref, xa_ref, xb_ref):
    x = x_ref[...]
    xa_ref[...] = jnp.dot(x, waT_ref[...], preferred_element_type=F32)
    xb_ref[...] = jnp.dot(x, wbT_ref[...], preferred_element_type=F32)


def _tc_pre(x, waT, wbT):
    bn = 2000
    grid = N // bn
    return pl.pallas_call(
        _pre_body,
        grid=(grid,),
        in_specs=[
            pl.BlockSpec((bn, D), lambda i: (i, 0)),
            pl.BlockSpec((D, D), lambda i: (0, 0)),
            pl.BlockSpec((D, D), lambda i: (0, 0)),
        ],
        out_specs=[
            pl.BlockSpec((bn, D), lambda i: (i, 0)),
            pl.BlockSpec((bn, D), lambda i: (i, 0)),
        ],
        out_shape=[
            jax.ShapeDtypeStruct((N, D), F32),
            jax.ShapeDtypeStruct((N, D), F32),
        ],
    )(x, waT, wbT)


def _ring(nch, process, issue):
    issue(0, 0)
    issue(1, 1)

    def step(i, carry):
        jj = i * 2
        for b in range(2):
            j = jj + b
            process(j, b)

            @pl.when(j + 2 < nch)
            def _():
                issue(j + 2, b)
        return carry

    if nch % 2 == 0:
        lax.fori_loop(0, (nch - 2) // 2, step, 0)
        process(nch - 2, 0)
        process(nch - 1, 1)
    else:
        lax.fori_loop(0, (nch - 1) // 2, step, 0)
        process(nch - 1, 0)


def _make_gather_body(ews, nch):
    def body(xa_hbm, xb_hbm, px_hbm, py_hbm, pz_hbm, row_hbm, col_hbm,
             s_hbm, d2_hbm,
             row_v, col_v, px_v, py_v, pz_v,
             bufa0, bufb0, bufa1, bufb1, spk0, spk1, d2_v,
             sa0, sb0, sa1, sb1):
        c = lax.axis_index("c")
        s = lax.axis_index("s")
        wid = s * NCORES + c
        base = wid * ews
        spk = (spk0, spk1)

        pltpu.sync_copy(row_hbm.at[wid], row_v)
        pltpu.sync_copy(col_hbm.at[wid], col_v)
        pltpu.sync_copy(px_hbm, px_v)
        pltpu.sync_copy(py_hbm, py_v)
        pltpu.sync_copy(pz_hbm, pz_v)

        bufs = ((bufa0, bufb0, sa0, sb0), (bufa1, bufb1, sa1, sb1))

        def issue(j, b):
            ba, bb, sa, sb = bufs[b]
            pltpu.async_copy(xa_hbm.at[col_v.at[j]], ba, sa)
            pltpu.async_copy(xb_hbm.at[row_v.at[j]], bb, sb)

        def process(j, b):
            ba, bb, sa, sb = bufs[b]
            sp = spk[b]
            for l in range(CB // 16):
                ir = row_v[j, pl.ds(l * 16, 16)]
                ic = col_v[j, pl.ds(l * 16, 16)]
                dx = plsc.load_gather(px_v, [ic]) - plsc.load_gather(px_v, [ir])
                dy = plsc.load_gather(py_v, [ic]) - plsc.load_gather(py_v, [ir])
                dz = plsc.load_gather(pz_v, [ic]) - plsc.load_gather(pz_v, [ir])
                d2_v[pl.ds(j * CB + l * 16, 16)] = dx * dx + dy * dy + dz * dz
            pltpu.make_async_copy(xa_hbm.at[col_v.at[j]], ba, sa).wait()
            pltpu.make_async_copy(xb_hbm.at[row_v.at[j]], bb, sb).wait()

            def addrow(r, cr):
                for m in range(D // 32):
                    s0 = pl.ds(32 * m, 16)
                    s1 = pl.ds(32 * m + 16, 16)
                    v0 = ba[r, s0] + bb[r, s0]
                    v1 = ba[r, s1] + bb[r, s1]
                    pk = plsc.pack(v0, v1, format=plsc.PackFormat.INTERLEAVED)
                    sp[r, pl.ds(16 * m, 16)] = plsc.bitcast(pk, jnp.int32)
                return cr

            lax.fori_loop(0, CB, addrow, 0)
            pltpu.sync_copy(sp, s_hbm.at[pl.ds(base + j * CB, CB)])

        _ring(nch, process, issue)
        pltpu.sync_copy(d2_v, d2_hbm.at[pl.ds(base, ews)])

    return body


def _sc_gather(xa, xb, px, py, pz, row3, col3):
    nw, nch, cb = row3.shape
    ews = nch * cb
    es = nw * ews
    mesh = plsc.VectorSubcoreMesh(
        core_axis_name="c", subcore_axis_name="s",
        num_cores=NCORES, num_subcores=NSUB)
    f = pl.kernel(
        _make_gather_body(ews, nch),
        out_type=(
            jax.ShapeDtypeStruct((es, D // 2), jnp.int32),
            jax.ShapeDtypeStruct((es,), F32),
        ),
        mesh=mesh,
        compiler_params=pltpu.CompilerParams(needs_layout_passes=False),
        scratch_types=[
            pltpu.VMEM((nch, CB), jnp.int32),
            pltpu.VMEM((nch, CB), jnp.int32),
            pltpu.VMEM((N,), F32),
            pltpu.VMEM((N,), F32),
            pltpu.VMEM((N,), F32),
            pltpu.VMEM((CB, D), F32),
            pltpu.VMEM((CB, D), F32),
            pltpu.VMEM((CB, D), F32),
            pltpu.VMEM((CB, D), F32),
            pltpu.VMEM((CB, D // 2), jnp.int32),
            pltpu.VMEM((CB, D // 2), jnp.int32),
            pltpu.VMEM((ews,), F32),
            pltpu.SemaphoreType.DMA,
            pltpu.SemaphoreType.DMA,
            pltpu.SemaphoreType.DMA,
            pltpu.SemaphoreType.DMA,
        ],
    )
    return f(xa, xb, px, py, pz, row3, col3)


def _edge_body(s_ref, d2_ref, ea_ref, wd_ref, bm1_ref, weT_ref, w2T_ref,
               bm2_ref, msg_ref):
    d2 = d2_ref[...]
    dist = jnp.clip(jnp.sqrt(d2 + 1e-12), 1e-4, 100.0)
    h = (s_ref[...].astype(F32) + dist * wd_ref[...]
         + jnp.dot(ea_ref[...], weT_ref[...], preferred_element_type=F32)
         + bm1_ref[...])
    h = h * jax.nn.sigmoid(h)
    m = jnp.dot(h.astype(BF16), w2T_ref[...],
                preferred_element_type=F32) + bm2_ref[...]
    msg_ref[...] = m * jax.nn.sigmoid(m)


def _tc_edge(s, d2c, ea, wd, bm1r, weT, w2T, bm2r):
    be = 2000
    es = s.shape[0]
    grid = es // be
    ed = ea.shape[1]
    return pl.pallas_call(
        _edge_body,
        grid=(grid,),
        in_specs=[
            pl.BlockSpec((be, D), lambda i: (i, 0)),
            pl.BlockSpec((be, 1), lambda i: (i, 0)),
            pl.BlockSpec((be, ed), lambda i: (i, 0)),
            pl.BlockSpec((1, D), lambda i: (0, 0)),
            pl.BlockSpec((1, D), lambda i: (0, 0)),
            pl.BlockSpec((ed, D), lambda i: (0, 0)),
            pl.BlockSpec((D, D), lambda i: (0, 0)),
            pl.BlockSpec((1, D), lambda i: (0, 0)),
        ],
        out_specs=pl.BlockSpec((be, D), lambda i: (i, 0)),
        out_shape=jax.ShapeDtypeStruct((es, D), F32),
    )(s, d2c, ea, wd, bm1r, weT, w2T, bm2r)


def _make_scatter_body(ews, nch):
    def body(msg_hbm, col_hbm, out_hbm, col_v, msg_v0, msg_v1, acc, sm0, sm1):
        c = lax.axis_index("c")
        s = lax.axis_index("s")
        wid = s * NCORES + c
        base = wid * ews

        def zrow(r, cr):
            for cc in range(D // 16):
                msg_v0[r, pl.ds(cc * 16, 16)] = jnp.zeros((16,), F32)
            return cr

        lax.fori_loop(0, CB, zrow, 0)
        for z in range(RPT // CB):
            pltpu.sync_copy(msg_v0, acc.at[pl.ds(s * RPT + z * CB, CB)])
        plsc.subcore_barrier()

        pltpu.sync_copy(col_hbm.at[wid], col_v)

        bufs = ((msg_v0, sm0), (msg_v1, sm1))

        def issue(j, b):
            mv, sm = bufs[b]
            pltpu.async_copy(msg_hbm.at[pl.ds(base + j * CB, CB)], mv, sm)

        def process(j, b):
            mv, sm = bufs[b]
            pltpu.make_async_copy(msg_hbm.at[pl.ds(base + j * CB, CB)], mv,
                                  sm).wait()
            pltpu.sync_copy(mv, acc.at[col_v.at[j]], add=True)

        _ring(nch, process, issue)
        plsc.subcore_barrier()
        pltpu.sync_copy(acc.at[pl.ds(s * RPT, RPT)],
                        out_hbm.at[c].at[pl.ds(s * RPT, RPT)])

    return body


def _sc_scatter(msg, col3):
    nw, nch, cb = col3.shape
    ews = nch * cb
    mesh = plsc.VectorSubcoreMesh(
        core_axis_name="c", subcore_axis_name="s",
        num_cores=NCORES, num_subcores=NSUB)
    f = pl.kernel(
        _make_scatter_body(ews, nch),
        out_type=jax.ShapeDtypeStruct((NCORES, NPAD, D), F32),
        mesh=mesh,
        compiler_params=pltpu.CompilerParams(needs_layout_passes=False),
        scratch_types=[
            pltpu.VMEM((nch, CB), jnp.int32),
            pltpu.VMEM((CB, D), F32),
            pltpu.VMEM((CB, D), F32),
            pltpu.VMEM_SHARED((NPAD, D), F32),
            pltpu.SemaphoreType.DMA,
            pltpu.SemaphoreType.DMA,
        ],
    )
    return f(msg, col3)


def _node_body(x_ref, p0_ref, p1_ref, wu1xT_ref, wu1aT_ref, bu1_ref, wu2T_ref,
               bu2_ref, wkT_ref, bk_ref, wvT_ref, bv_ref, qbd_ref,
               xn_ref, v_ref, ae_ref):
    agg = p0_ref[0] + p0_ref[1] + p1_ref[0] + p1_ref[1]
    h = (jnp.dot(x_ref[...], wu1xT_ref[...], preferred_element_type=F32)
         + jnp.dot(agg, wu1aT_ref[...], preferred_element_type=F32)
         + bu1_ref[...])
    h = h * jax.nn.sigmoid(h)
    xn = jnp.dot(h, wu2T_ref[...], preferred_element_type=F32) + bu2_ref[...]
    xn_ref[...] = xn
    k = jnp.dot(xn, wkT_ref[...], preferred_element_type=F32) + bk_ref[...]
    v_ref[...] = jnp.dot(xn, wvT_ref[...], preferred_element_type=F32) + bv_ref[...]
    attn = jnp.clip(jnp.dot(k, qbd_ref[...], preferred_element_type=F32)
                    * (HDIM ** -0.5), -20.0, 20.0)
    ae_ref[...] = jnp.exp(attn)


def _tc_node(x, p0, p1, wu1xT, wu1aT, bu1r, wu2T, bu2r, wkT, bkr, wvT, bvr,
             qbd):
    bn = 2000
    grid = N // bn
    return pl.pallas_call(
        _node_body,
        grid=(grid,),
        in_specs=[
            pl.BlockSpec((bn, D), lambda i: (i, 0)),
            pl.BlockSpec((NCORES, bn, D), lambda i: (0, i, 0)),
            pl.BlockSpec((NCORES, bn, D), lambda i: (0, i, 0)),
            pl.BlockSpec((D, D), lambda i: (0, 0)),
            pl.BlockSpec((D, D), lambda i: (0, 0)),
            pl.BlockSpec((1, D), lambda i: (0, 0)),
            pl.BlockSpec((D, D), lambda i: (0, 0)),
            pl.BlockSpec((1, D), lambda i: (0, 0)),
            pl.BlockSpec((D, D), lambda i: (0, 0)),
            pl.BlockSpec((1, D), lambda i: (0, 0)),
            pl.BlockSpec((D, D), lambda i: (0, 0)),
            pl.BlockSpec((1, D), lambda i: (0, 0)),
            pl.BlockSpec((D, NHEADS), lambda i: (0, 0)),
        ],
        out_specs=[
            pl.BlockSpec((bn, D), lambda i: (i, 0)),
            pl.BlockSpec((bn, D), lambda i: (i, 0)),
            pl.BlockSpec((bn, NHEADS), lambda i: (i, 0)),
        ],
        out_shape=[
            jax.ShapeDtypeStruct((N, D), F32),
            jax.ShapeDtypeStruct((N, D), F32),
            jax.ShapeDtypeStruct((N, NHEADS), F32),
        ],
    )(x, p0, p1, wu1xT, wu1aT, bu1r, wu2T, bu2r, wkT, bkr, wvT, bvr, qbd)


def _tail_body(xn_ref, v_ref, ae_ref, b2_ref, bT_ref, wihT_ref, whhT_ref,
               bih_ref, bhh_ref, out_ref):
    xn = xn_ref[...]
    v = v_ref[...]
    ae = ae_ref[...]
    oh = (b2_ref[...] == lax.broadcasted_iota(jnp.int32, (N, NGRAPH), 1)
          ).astype(F32)
    ohT = (bT_ref[...] == lax.broadcasted_iota(jnp.int32, (NGRAPH, N), 0)
           ).astype(F32)

    asum = jnp.dot(ohT, ae, preferred_element_type=F32)
    cols = []
    for hh in range(NHEADS):
        u = ae[:, hh:hh + 1] * v[:, HDIM * hh:HDIM * (hh + 1)]
        num = jnp.dot(ohT, u, preferred_element_type=F32)
        cols.append(num / (asum[:, hh:hh + 1] + 1e-8))
    pooled = jnp.concatenate(cols, axis=1)

    hs = jnp.zeros((NGRAPH, D), F32)
    cs = jnp.zeros((NGRAPH, D), F32)
    qs = jnp.zeros((NGRAPH, 2 * D), F32)
    for _ in range(3):
        gates = (jnp.dot(qs, wihT_ref[...], preferred_element_type=F32)
                 + bih_ref[...]
                 + jnp.dot(hs, whhT_ref[...], preferred_element_type=F32)
                 + bhh_ref[...])
        ig = jax.nn.sigmoid(gates[:, :D])
        fg = jax.nn.sigmoid(gates[:, D:2 * D])
        gg = jnp.tanh(gates[:, 2 * D:3 * D])
        og = jax.nn.sigmoid(gates[:, 3 * D:4 * D])
        cs = fg * cs + ig * gg
        hs = og * jnp.tanh(cs)
        qn = jnp.dot(oh, hs, preferred_element_type=F32)
        a = jnp.clip(jnp.sum(xn * qn, axis=1, keepdims=True), -20.0, 20.0)
        ae2 = jnp.exp(a)
        s2 = jnp.dot(ohT, ae2, preferred_element_type=F32)
        r = jnp.dot(ohT, ae2 * xn, preferred_element_type=F32) / (s2 + 1e-8)
        qs = jnp.concatenate([hs, r], axis=1)
    out_ref[...] = jnp.concatenate([pooled, qs], axis=1)


def _tc_tail(xn, v, ae, b2, bT, wihT, whhT, bihr, bhhr):
    return pl.pallas_call(
        _tail_body,
        out_shape=jax.ShapeDtypeStruct((NGRAPH, 3 * D), F32),
    )(xn, v, ae, b2, bT, wihT, whhT, bihr, bhhr)


def kernel(x, pos, edge_index, edge_attr, batch, Wm1, bm1, Wm2, bm2,
           Wu1, bu1, Wu2, bu2, Wp1, bp1, Wp2, bp2, q_ap, Wk, bk, Wv, bv,
           Wih, Whh, bih, bhh):
    row = edge_index[0]
    col = edge_index[1]
    bounds = []
    off = 0
    for es in SLICES:
        bounds.append((off, off + es))
        off += es
    row4 = [row[a:b].reshape(NW, (b - a) // (NW * CB), CB) for a, b in bounds]
    col4 = [col[a:b].reshape(NW, (b - a) // (NW * CB), CB) for a, b in bounds]
    eab = edge_attr.astype(BF16)
    ea3 = [eab[a:b] for a, b in bounds]
    px = pos[:, 0]
    py = pos[:, 1]
    pz = pos[:, 2]

    waT = Wm1[:, :D].T
    wbT = Wm1[:, D:2 * D].T
    wd = Wm1[:, 2 * D].reshape(1, D)
    weT = Wm1[:, 2 * D + 1:].T.astype(BF16)
    w2T = Wm2.T.astype(BF16)
    bm1r = bm1.reshape(1, D)
    bm2r = bm2.reshape(1, D)

    qbd = jnp.zeros((D, NHEADS), F32)
    for hh in range(NHEADS):
        qbd = qbd.at[HDIM * hh:HDIM * (hh + 1), hh].set(q_ap[hh])

    wd_p = wd[:, _SIGMA]
    bm1_p = bm1r[:, _SIGMA]
    weT_p = weT[:, _SIGMA]
    w2T_p = w2T[_SIGMA, :]

    xa, xb = _tc_pre(x, waT, wbT)
    parts = []
    for si in range(len(SLICES)):
        s32, d2 = _sc_gather(xa, xb, px, py, pz, row4[si], col4[si])
        s = lax.bitcast_convert_type(s32, BF16).reshape(-1, D)
        msg = _tc_edge(s, d2.reshape(-1, 1), ea3[si], wd_p, bm1_p, weT_p,
                       w2T_p, bm2r)
        parts.append(_sc_scatter(msg, col4[si]))
    xn, v, ae = _tc_node(
        x, parts[0][:, :N, :], parts[1][:, :N, :],
        Wu1[:, :D].T, Wu1[:, D:].T, bu1.reshape(1, D),
        Wu2.T, bu2.reshape(1, D), Wk.T, bk.reshape(1, D), Wv.T,
        bv.reshape(1, D), qbd)
    out = _tc_tail(
        xn, v, ae, batch.reshape(N, 1), batch.reshape(1, N),
        Wih.T, Whh.T, bih.reshape(1, 4 * D), bhh.reshape(1, 4 * D))
    return out


# speedup vs baseline: 2.0457x; 2.0457x over previous
import jax
import jax.numpy as jnp
import numpy as np
from jax import lax
from jax.experimental import pallas as pl
from jax.experimental.pallas import tpu as pltpu
from jax.experimental.pallas import tpu_sc as plsc

F32 = jnp.float32
BF16 = jnp.bfloat16

N = 10000
E = 320000
D = 128
NHEADS = 4
HDIM = 32
NGRAPH = 64

NCORES = 2
NSUB = 16
NW = NCORES * NSUB
SLICES = (192000, 128000)
CB = 80
NPAD = 10240
RPT = NPAD // NSUB

_SIGMA = np.zeros(D, np.int32)
for _m in range(D // 32):
    for _i in range(16):
        for _b in range(2):
            _SIGMA[32 * _m + 2 * _i + _b] = 32 * _m + 16 * _b + _i


def _pre_body(x_ref, waT_ref, wbT_ref, xa_ref, xb_ref):
    x = x_ref[...]
    xa_ref[...] = jnp.dot(x, waT_ref[...], preferred_element_type=F32)
    xb_ref[...] = jnp.dot(x, wbT_ref[...], preferred_element_type=F32)


def _tc_pre(x, waT, wbT):
    bn = 2000
    grid = N // bn
    return pl.pallas_call(
        _pre_body,
        grid=(grid,),
        in_specs=[
            pl.BlockSpec((bn, D), lambda i: (i, 0)),
            pl.BlockSpec((D, D), lambda i: (0, 0)),
            pl.BlockSpec((D, D), lambda i: (0, 0)),
        ],
        out_specs=[
            pl.BlockSpec((bn, D), lambda i: (i, 0)),
            pl.BlockSpec((bn, D), lambda i: (i, 0)),
        ],
        out_shape=[
            jax.ShapeDtypeStruct((N, D), F32),
            jax.ShapeDtypeStruct((N, D), F32),
        ],
    )(x, waT, wbT)


def _ring(nch, process, issue):
    issue(0, 0)
    issue(1, 1)

    def step(i, carry):
        jj = i * 2
        for b in range(2):
            j = jj + b
            process(j, b)

            @pl.when(j + 2 < nch)
            def _():
                issue(j + 2, b)
        return carry

    if nch % 2 == 0:
        lax.fori_loop(0, (nch - 2) // 2, step, 0)
        process(nch - 2, 0)
        process(nch - 1, 1)
    else:
        lax.fori_loop(0, (nch - 1) // 2, step, 0)
        process(nch - 1, 0)


def _make_gather_body(ews, nch):
    def body(xa_hbm, xb_hbm, px_hbm, py_hbm, pz_hbm, row_hbm, col_hbm,
             s_hbm, d2_hbm,
             row_v, col_v, px_v, py_v, pz_v,
             bufa0, bufb0, bufa1, bufb1, d2_v,
             sa0, sb0, sa1, sb1):
        c = lax.axis_index("c")
        s = lax.axis_index("s")
        wid = s * NCORES + c
        base = wid * ews

        pltpu.sync_copy(row_hbm.at[wid], row_v)
        pltpu.sync_copy(col_hbm.at[wid], col_v)
        pltpu.sync_copy(px_hbm, px_v)
        pltpu.sync_copy(py_hbm, py_v)
        pltpu.sync_copy(pz_hbm, pz_v)

        bufs = ((bufa0, bufb0, sa0, sb0), (bufa1, bufb1, sa1, sb1))

        def issue(j, b):
            ba, bb, sa, sb = bufs[b]
            pltpu.async_copy(xa_hbm.at[col_v.at[j]], ba, sa)
            pltpu.async_copy(xb_hbm.at[row_v.at[j]], bb, sb)

        def process(j, b):
            ba, bb, sa, sb = bufs[b]
            for l in range(CB // 16):
                ir = row_v[j, pl.ds(l * 16, 16)]
                ic = col_v[j, pl.ds(l * 16, 16)]
                dx = plsc.load_gather(px_v, [ic]) - plsc.load_gather(px_v, [ir])
                dy = plsc.load_gather(py_v, [ic]) - plsc.load_gather(py_v, [ir])
                dz = plsc.load_gather(pz_v, [ic]) - plsc.load_gather(pz_v, [ir])
                d2_v[pl.ds(j * CB + l * 16, 16)] = dx * dx + dy * dy + dz * dz
            pltpu.make_async_copy(xa_hbm.at[col_v.at[j]], ba, sa).wait()
            pltpu.make_async_copy(xb_hbm.at[row_v.at[j]], bb, sb).wait()

            def addrow(r, cr):
                for cc in range(D // 16):
                    sl = pl.ds(cc * 16, 16)
                    ba[r, sl] = ba[r, sl] + bb[r, sl]
                return cr

            lax.fori_loop(0, CB, addrow, 0)
            pltpu.sync_copy(ba, s_hbm.at[pl.ds(base + j * CB, CB)])

        _ring(nch, process, issue)
        pltpu.sync_copy(d2_v, d2_hbm.at[pl.ds(base, ews)])

    return body


def _sc_gather(xa, xb, px, py, pz, row3, col3):
    nw, nch, cb = row3.shape
    ews = nch * cb
    es = nw * ews
    mesh = plsc.VectorSubcoreMesh(
        core_axis_name="c", subcore_axis_name="s",
        num_cores=NCORES, num_subcores=NSUB)
    f = pl.kernel(
        _make_gather_body(ews, nch),
        out_type=(
            jax.ShapeDtypeStruct((es, D), F32),
            jax.ShapeDtypeStruct((es,), F32),
        ),
        mesh=mesh,
        compiler_params=pltpu.CompilerParams(needs_layout_passes=False),
        scratch_types=[
            pltpu.VMEM((nch, CB), jnp.int32),
            pltpu.VMEM((nch, CB), jnp.int32),
            pltpu.VMEM((N,), F32),
            pltpu.VMEM((N,), F32),
            pltpu.VMEM((N,), F32),
            pltpu.VMEM((CB, D), F32),
            pltpu.VMEM((CB, D), F32),
            pltpu.VMEM((CB, D), F32),
            pltpu.VMEM((CB, D), F32),
            pltpu.VMEM((ews,), F32),
            pltpu.SemaphoreType.DMA,
            pltpu.SemaphoreType.DMA,
            pltpu.SemaphoreType.DMA,
            pltpu.SemaphoreType.DMA,
        ],
    )
    return f(xa, xb, px, py, pz, row3, col3)


def _edge_body(s_ref, d2_ref, ea_ref, wd_ref, bm1_ref, weT_ref, w2T_ref,
               bm2_ref, msg_ref):
    d2 = d2_ref[...]
    dist = jnp.clip(jnp.sqrt(d2 + 1e-12), 1e-4, 100.0)
    h = (s_ref[...] + dist * wd_ref[...]
         + jnp.dot(ea_ref[...], weT_ref[...], preferred_element_type=F32)
         + bm1_ref[...])
    h = h * jax.nn.sigmoid(h)
    m = jnp.dot(h.astype(BF16), w2T_ref[...],
                preferred_element_type=F32) + bm2_ref[...]
    msg_ref[...] = m * jax.nn.sigmoid(m)


def _tc_edge(s, d2c, ea, wd, bm1r, weT, w2T, bm2r):
    be = 2000
    es = s.shape[0]
    grid = es // be
    ed = ea.shape[1]
    return pl.pallas_call(
        _edge_body,
        grid=(grid,),
        in_specs=[
            pl.BlockSpec((be, D), lambda i: (i, 0)),
            pl.BlockSpec((be, 1), lambda i: (i, 0)),
            pl.BlockSpec((be, ed), lambda i: (i, 0)),
            pl.BlockSpec((1, D), lambda i: (0, 0)),
            pl.BlockSpec((1, D), lambda i: (0, 0)),
            pl.BlockSpec((ed, D), lambda i: (0, 0)),
            pl.BlockSpec((D, D), lambda i: (0, 0)),
            pl.BlockSpec((1, D), lambda i: (0, 0)),
        ],
        out_specs=pl.BlockSpec((be, D), lambda i: (i, 0)),
        out_shape=jax.ShapeDtypeStruct((es, D), F32),
    )(s, d2c, ea, wd, bm1r, weT, w2T, bm2r)


def _make_scatter_body(ews, nch):
    def body(msg_hbm, col_hbm, out_hbm, col_v, msg_v0, msg_v1, acc, sm0, sm1):
        c = lax.axis_index("c")
        s = lax.axis_index("s")
        wid = s * NCORES + c
        base = wid * ews

        def zrow(r, cr):
            for cc in range(D // 16):
                msg_v0[r, pl.ds(cc * 16, 16)] = jnp.zeros((16,), F32)
            return cr

        lax.fori_loop(0, CB, zrow, 0)
        for z in range(RPT // CB):
            pltpu.sync_copy(msg_v0, acc.at[pl.ds(s * RPT + z * CB, CB)])
        plsc.subcore_barrier()

        pltpu.sync_copy(col_hbm.at[wid], col_v)

        bufs = ((msg_v0, sm0), (msg_v1, sm1))

        def issue(j, b):
            mv, sm = bufs[b]
            pltpu.async_copy(msg_hbm.at[pl.ds(base + j * CB, CB)], mv, sm)

        def process(j, b):
            mv, sm = bufs[b]
            pltpu.make_async_copy(msg_hbm.at[pl.ds(base + j * CB, CB)], mv,
                                  sm).wait()
            pltpu.sync_copy(mv, acc.at[col_v.at[j]], add=True)

        _ring(nch, process, issue)
        plsc.subcore_barrier()
        pltpu.sync_copy(acc.at[pl.ds(s * RPT, RPT)],
                        out_hbm.at[c].at[pl.ds(s * RPT, RPT)])

    return body


def _sc_scatter(msg, col3):
    nw, nch, cb = col3.shape
    ews = nch * cb
    mesh = plsc.VectorSubcoreMesh(
        core_axis_name="c", subcore_axis_name="s",
        num_cores=NCORES, num_subcores=NSUB)
    f = pl.kernel(
        _make_scatter_body(ews, nch),
        out_type=jax.ShapeDtypeStruct((NCORES, NPAD, D), F32),
        mesh=mesh,
        compiler_params=pltpu.CompilerParams(needs_layout_passes=False),
        scratch_types=[
            pltpu.VMEM((nch, CB), jnp.int32),
            pltpu.VMEM((CB, D), F32),
            pltpu.VMEM((CB, D), F32),
            pltpu.VMEM_SHARED((NPAD, D), F32),
            pltpu.SemaphoreType.DMA,
            pltpu.SemaphoreType.DMA,
        ],
    )
    return f(msg, col3)


def _node_body(x_ref, p0_ref, p1_ref, wu1xT_ref, wu1aT_ref, bu1_ref, wu2T_ref,
               bu2_ref, wkT_ref, bk_ref, wvT_ref, bv_ref, qbd_ref,
               xn_ref, v_ref, ae_ref):
    agg = p0_ref[0] + p0_ref[1] + p1_ref[0] + p1_ref[1]
    h = (jnp.dot(x_ref[...], wu1xT_ref[...], preferred_element_type=F32)
         + jnp.dot(agg, wu1aT_ref[...], preferred_element_type=F32)
         + bu1_ref[...])
    h = h * jax.nn.sigmoid(h)
    xn = jnp.dot(h, wu2T_ref[...], preferred_element_type=F32) + bu2_ref[...]
    xn_ref[...] = xn
    k = jnp.dot(xn, wkT_ref[...], preferred_element_type=F32) + bk_ref[...]
    v_ref[...] = jnp.dot(xn, wvT_ref[...], preferred_element_type=F32) + bv_ref[...]
    attn = jnp.clip(jnp.dot(k, qbd_ref[...], preferred_element_type=F32)
                    * (HDIM ** -0.5), -20.0, 20.0)
    ae_ref[...] = jnp.exp(attn)


def _tc_node(x, p0, p1, wu1xT, wu1aT, bu1r, wu2T, bu2r, wkT, bkr, wvT, bvr,
             qbd):
    bn = 2000
    grid = N // bn
    return pl.pallas_call(
        _node_body,
        grid=(grid,),
        in_specs=[
            pl.BlockSpec((bn, D), lambda i: (i, 0)),
            pl.BlockSpec((NCORES, bn, D), lambda i: (0, i, 0)),
            pl.BlockSpec((NCORES, bn, D), lambda i: (0, i, 0)),
            pl.BlockSpec((D, D), lambda i: (0, 0)),
            pl.BlockSpec((D, D), lambda i: (0, 0)),
            pl.BlockSpec((1, D), lambda i: (0, 0)),
            pl.BlockSpec((D, D), lambda i: (0, 0)),
            pl.BlockSpec((1, D), lambda i: (0, 0)),
            pl.BlockSpec((D, D), lambda i: (0, 0)),
            pl.BlockSpec((1, D), lambda i: (0, 0)),
            pl.BlockSpec((D, D), lambda i: (0, 0)),
            pl.BlockSpec((1, D), lambda i: (0, 0)),
            pl.BlockSpec((D, NHEADS), lambda i: (0, 0)),
        ],
        out_specs=[
            pl.BlockSpec((bn, D), lambda i: (i, 0)),
            pl.BlockSpec((bn, D), lambda i: (i, 0)),
            pl.BlockSpec((bn, NHEADS), lambda i: (i, 0)),
        ],
        out_shape=[
            jax.ShapeDtypeStruct((N, D), F32),
            jax.ShapeDtypeStruct((N, D), F32),
            jax.ShapeDtypeStruct((N, NHEADS), F32),
        ],
    )(x, p0, p1, wu1xT, wu1aT, bu1r, wu2T, bu2r, wkT, bkr, wvT, bvr, qbd)


def _tail_body(xn_ref, v_ref, ae_ref, b2_ref, bT_ref, wihT_ref, whhT_ref,
               bih_ref, bhh_ref, out_ref):
    xn = xn_ref[...]
    v = v_ref[...]
    ae = ae_ref[...]
    oh = (b2_ref[...] == lax.broadcasted_iota(jnp.int32, (N, NGRAPH), 1)
          ).astype(F32)
    ohT = (bT_ref[...] == lax.broadcasted_iota(jnp.int32, (NGRAPH, N), 0)
           ).astype(F32)

    asum = jnp.dot(ohT, ae, preferred_element_type=F32)
    cols = []
    for hh in range(NHEADS):
        u = ae[:, hh:hh + 1] * v[:, HDIM * hh:HDIM * (hh + 1)]
        num = jnp.dot(ohT, u, preferred_element_type=F32)
        cols.append(num / (asum[:, hh:hh + 1] + 1e-8))
    pooled = jnp.concatenate(cols, axis=1)

    hs = jnp.zeros((NGRAPH, D), F32)
    cs = jnp.zeros((NGRAPH, D), F32)
    qs = jnp.zeros((NGRAPH, 2 * D), F32)
    for _ in range(3):
        gates = (jnp.dot(qs, wihT_ref[...], preferred_element_type=F32)
                 + bih_ref[...]
                 + jnp.dot(hs, whhT_ref[...], preferred_element_type=F32)
                 + bhh_ref[...])
        ig = jax.nn.sigmoid(gates[:, :D])
        fg = jax.nn.sigmoid(gates[:, D:2 * D])
        gg = jnp.tanh(gates[:, 2 * D:3 * D])
        og = jax.nn.sigmoid(gates[:, 3 * D:4 * D])
        cs = fg * cs + ig * gg
        hs = og * jnp.tanh(cs)
        qn = jnp.dot(oh, hs, preferred_element_type=F32)
        a = jnp.clip(jnp.sum(xn * qn, axis=1, keepdims=True), -20.0, 20.0)
        ae2 = jnp.exp(a)
        s2 = jnp.dot(ohT, ae2, preferred_element_type=F32)
        r = jnp.dot(ohT, ae2 * xn, preferred_element_type=F32) / (s2 + 1e-8)
        qs = jnp.concatenate([hs, r], axis=1)
    out_ref[...] = jnp.concatenate([pooled, qs], axis=1)


def _tc_tail(xn, v, ae, b2, bT, wihT, whhT, bihr, bhhr):
    return pl.pallas_call(
        _tail_body,
        out_shape=jax.ShapeDtypeStruct((NGRAPH, 3 * D), F32),
    )(xn, v, ae, b2, bT, wihT, whhT, bihr, bhhr)


def kernel(x, pos, edge_index, edge_attr, batch, Wm1, bm1, Wm2, bm2,
           Wu1, bu1, Wu2, bu2, Wp1, bp1, Wp2, bp2, q_ap, Wk, bk, Wv, bv,
           Wih, Whh, bih, bhh):
    row = edge_index[0]
    col = edge_index[1]
    bounds = []
    off = 0
    for es in SLICES:
        bounds.append((off, off + es))
        off += es
    row4 = [row[a:b].reshape(NW, (b - a) // (NW * CB), CB) for a, b in bounds]
    col4 = [col[a:b].reshape(NW, (b - a) // (NW * CB), CB) for a, b in bounds]
    eab = edge_attr.astype(BF16)
    ea3 = [eab[a:b] for a, b in bounds]
    px = pos[:, 0]
    py = pos[:, 1]
    pz = pos[:, 2]

    waT = Wm1[:, :D].T
    wbT = Wm1[:, D:2 * D].T
    wd = Wm1[:, 2 * D].reshape(1, D)
    weT = Wm1[:, 2 * D + 1:].T.astype(BF16)
    w2T = Wm2.T.astype(BF16)
    bm1r = bm1.reshape(1, D)
    bm2r = bm2.reshape(1, D)

    qbd = jnp.zeros((D, NHEADS), F32)
    for hh in range(NHEADS):
        qbd = qbd.at[HDIM * hh:HDIM * (hh + 1), hh].set(q_ap[hh])

    xa, xb = _tc_pre(x, waT, wbT)
    parts = []
    for si in range(len(SLICES)):
        s, d2 = _sc_gather(xa, xb, px, py, pz, row4[si], col4[si])
        msg = _tc_edge(s, d2.reshape(-1, 1), ea3[si], wd, bm1r, weT, w2T,
                       bm2r)
        parts.append(_sc_scatter(msg, col4[si]))
    xn, v, ae = _tc_node(
        x, parts[0][:, :N, :], parts[1][:, :N, :],
        Wu1[:, :D].T, Wu1[:, D:].T, bu1.reshape(1, D),
        Wu2.T, bu2.reshape(1, D), Wk.T, bk.reshape(1, D), Wv.T,
        bv.reshape(1, D), qbd)
    out = _tc_tail(
        xn, v, ae, batch.reshape(N, 1), batch.reshape(1, N),
        Wih.T, Whh.T, bih.reshape(1, 4 * D), bhh.reshape(1, 4 * D))
    return out
